# Initial kernel scaffold; baseline (speedup 1.0000x reference)
import functools

import jax
import jax.numpy as jnp
from jax import lax
from jax.experimental import pallas as pl
from jax.experimental.pallas import tpu as pltpu
from jax.experimental.pallas import tpu_sc as plsc

N = 10000
NP = 10240
E = 160000
EP = 163840
D = 256
H = 128
K = 128
NSUB = 16
EDGES_PER_TILE = EP // NSUB
CHUNKS = EDGES_PER_TILE // K
ROWS_PER_TILE = NP // NSUB
RB = 1280
GRID = NP // RB


@functools.lru_cache(maxsize=None)
def _sc_kernels():
    mesh = plsc.VectorSubcoreMesh(core_axis_name="c", subcore_axis_name="s")

    @functools.partial(
        pl.kernel,
        out_type=(jax.ShapeDtypeStruct((NP, 16), jnp.int32),
                  jax.ShapeDtypeStruct((NP, 16), jnp.int32)),
        mesh=mesh,
        scratch_types=[
            pltpu.VMEM_SHARED((NP, 16), jnp.int32),
            pltpu.VMEM((K,), jnp.int32),
            pltpu.VMEM((K, 16), jnp.int32),
        ],
    )
    def degree_kernel(ei, z16, ones, cs_out, cd_out, cnt_sh, idx_v, ones_v):
        cid = lax.axis_index("c")
        sid = lax.axis_index("s")
        r0 = sid * ROWS_PER_TILE
        rows = pl.ds(r0, ROWS_PER_TILE)
        pltpu.sync_copy(z16.at[rows], cnt_sh.at[rows])
        pltpu.sync_copy(ones, ones_v)
        plsc.subcore_barrier()
        e0 = sid * EDGES_PER_TILE

        @pl.loop(0, CHUNKS)
        def _(j):
            pltpu.sync_copy(ei.at[cid, pl.ds(e0 + j * K, K)], idx_v)
            pltpu.sync_copy(ones_v, cnt_sh.at[idx_v], add=True)

        plsc.subcore_barrier()

        @pl.when(cid == 0)
        def _():
            pltpu.sync_copy(cnt_sh.at[rows], cs_out.at[rows])

        @pl.when(cid == 1)
        def _():
            pltpu.sync_copy(cnt_sh.at[rows], cd_out.at[rows])

    @functools.partial(
        pl.kernel,
        out_type=(jax.ShapeDtypeStruct((NP, H), jnp.float32),
                  jax.ShapeDtypeStruct((NP, H), jnp.float32)),
        mesh=mesh,
        scratch_types=[
            pltpu.VMEM_SHARED((NP, H), jnp.float32),
            pltpu.VMEM((K,), jnp.int32),
            pltpu.VMEM((K,), jnp.int32),
            pltpu.VMEM((K, H), jnp.float32),
        ],
    )
    def scatter_kernel(ei, el_lo, el_hi, zf, slo_out, shi_out,
                       acc_sh, src_v, dst_v, rows_v):
        cid = lax.axis_index("c")
        sid = lax.axis_index("s")
        r0 = sid * ROWS_PER_TILE
        rows = pl.ds(r0, ROWS_PER_TILE)
        pltpu.sync_copy(zf.at[rows], acc_sh.at[rows])
        plsc.subcore_barrier()
        e0 = sid * EDGES_PER_TILE

        @pl.loop(0, CHUNKS)
        def _(j):
            b = e0 + j * K
            pltpu.sync_copy(ei.at[0, pl.ds(b, K)], src_v)
            pltpu.sync_copy(ei.at[1, pl.ds(b, K)], dst_v)

            @pl.when(cid == 0)
            def _():
                pltpu.sync_copy(el_lo.at[src_v], rows_v)

            @pl.when(cid == 1)
            def _():
                pltpu.sync_copy(el_hi.at[src_v], rows_v)

            pltpu.sync_copy(rows_v, acc_sh.at[dst_v], add=True)

        plsc.subcore_barrier()

        @pl.when(cid == 0)
        def _():
            pltpu.sync_copy(acc_sh.at[rows], slo_out.at[rows])

        @pl.when(cid == 1)
        def _():
            pltpu.sync_copy(acc_sh.at[rows], shi_out.at[rows])

    return degree_kernel, scatter_kernel


def _scale_body(feat_ref, cnt_ref, lo_ref, hi_ref):
    cnt = cnt_ref[:, 0:1].astype(jnp.float32)
    norm = lax.rsqrt(jnp.maximum(cnt, 1.0))
    el = feat_ref[...] * norm
    lo_ref[...] = el[:, :H]
    hi_ref[...] = el[:, H:]


def _final_body(feat_ref, slo_ref, shi_ref, cnt_ref, wm_ref, wn_ref,
                bmn_ref, bn_ref, out_ref):
    cnt = cnt_ref[:, 0:1].astype(jnp.float32)
    norm = lax.rsqrt(jnp.maximum(cnt, 1.0))
    s = jnp.concatenate([slo_ref[...], shi_ref[...]], axis=1)
    t = norm * s
    f = feat_ref[...]
    out_ref[...] = (
        jnp.dot(t * f, wm_ref[...], preferred_element_type=jnp.float32)
        + jnp.dot(t + f, wn_ref[...], preferred_element_type=jnp.float32)
        + (norm * cnt) * bmn_ref[...]
        + bn_ref[...]
    )


def kernel(feat, edge_index, Wm, bm, Wn, bn):
    ei = jnp.concatenate(
        [edge_index.astype(jnp.int32), jnp.full((2, EP - E), N, jnp.int32)],
        axis=1)
    feat_p = jnp.pad(feat, ((0, NP - N), (0, 0)))
    z16 = jnp.zeros((NP, 16), jnp.int32)
    ones = jnp.ones((K, 16), jnp.int32)
    zf = jnp.zeros((NP, H), jnp.float32)

    degree_kernel, scatter_kernel = _sc_kernels()
    cs_w, cd_w = degree_kernel(ei, z16, ones)

    el_lo, el_hi = pl.pallas_call(
        _scale_body,
        grid=(GRID,),
        in_specs=[
            pl.BlockSpec((RB, D), lambda i: (i, 0)),
            pl.BlockSpec((RB, 16), lambda i: (i, 0)),
        ],
        out_specs=[
            pl.BlockSpec((RB, H), lambda i: (i, 0)),
            pl.BlockSpec((RB, H), lambda i: (i, 0)),
        ],
        out_shape=[
            jax.ShapeDtypeStruct((NP, H), jnp.float32),
            jax.ShapeDtypeStruct((NP, H), jnp.float32),
        ],
    )(feat_p, cs_w)

    s_lo, s_hi = scatter_kernel(ei, el_lo, el_hi, zf)

    out = pl.pallas_call(
        _final_body,
        grid=(GRID,),
        in_specs=[
            pl.BlockSpec((RB, D), lambda i: (i, 0)),
            pl.BlockSpec((RB, H), lambda i: (i, 0)),
            pl.BlockSpec((RB, H), lambda i: (i, 0)),
            pl.BlockSpec((RB, 16), lambda i: (i, 0)),
            pl.BlockSpec((D, D), lambda i: (0, 0)),
            pl.BlockSpec((D, D), lambda i: (0, 0)),
            pl.BlockSpec((1, D), lambda i: (0, 0)),
            pl.BlockSpec((1, D), lambda i: (0, 0)),
        ],
        out_specs=pl.BlockSpec((RB, D), lambda i: (i, 0)),
        out_shape=jax.ShapeDtypeStruct((N, D), jnp.float32),
    )(feat_p, s_lo, s_hi, cd_w, Wm, Wn,
      (bm + bn).reshape(1, D), bn.reshape(1, D))

    return out


# speedup vs baseline: 3.7550x; 3.7550x over previous
import dataclasses
import functools

import jax
import jax.numpy as jnp
from jax import lax
from jax.experimental import pallas as pl
from jax.experimental.pallas import tpu as pltpu
from jax.experimental.pallas import tpu_sc as plsc

N = 10000
NP = 10240
E = 160000
EP = 163840
D = 256
H = 128
K = 128
NSUB = 16
NCORE = 2
CROWS = NP // H
ROWS_PER_TILE = NP // NSUB
CROWS_PER_TILE = CROWS // NSUB
EPT = EP // NSUB
CHUNKS = EPT // K
RB = 1280
GRID = NP // RB


@functools.lru_cache(maxsize=None)
def _sc_kernels():
    mesh = plsc.VectorSubcoreMesh(core_axis_name="c", subcore_axis_name="s")
    cp = pltpu.CompilerParams()
    if "needs_layout_passes" in pltpu.CompilerParams.__dataclass_fields__:
        cp = dataclasses.replace(cp, needs_layout_passes=False)

    @functools.partial(
        pl.kernel,
        out_type=(jax.ShapeDtypeStruct((CROWS, H), jnp.int32),
                  jax.ShapeDtypeStruct((CROWS, H), jnp.int32)),
        mesh=mesh,
        compiler_params=cp,
        scratch_types=[
            pltpu.VMEM_SHARED((CROWS, H), jnp.int32),
            pltpu.VMEM((CROWS, H), jnp.int32),
            pltpu.VMEM((K,), jnp.int32),
            pltpu.VMEM((CROWS,), jnp.int32),
        ],
    )
    def degree_kernel(src_hbm, dst_hbm, z80, cs_out, cd_out,
                      acc_sh, cnt_v, idx_v, row_v):
        cid = lax.axis_index("c")
        sid = lax.axis_index("s")
        pltpu.sync_copy(z80, cnt_v)
        for k in range(CROWS // 16):
            row_v[pl.ds(k * 16, 16)] = lax.iota(jnp.int32, 16) + k * 16

        @pl.when(sid == 0)
        def _():
            pltpu.sync_copy(z80, acc_sh)

        ones16 = jnp.ones((16,), jnp.int32)
        e0 = sid * EPT

        @pl.loop(0, CHUNKS)
        def _(j):
            b = pl.ds(e0 + j * K, K)

            @pl.when(cid == 0)
            def _():
                pltpu.sync_copy(src_hbm.at[b], idx_v)

            @pl.when(cid == 1)
            def _():
                pltpu.sync_copy(dst_hbm.at[b], idx_v)

            for k in range(K // 16):
                v = idx_v[pl.ds(k * 16, 16)]
                hi = lax.shift_right_logical(v, 7)
                lo = lax.bitwise_and(v, 127)
                plsc.addupdate_scatter(cnt_v, [hi, lo], ones16)

        plsc.subcore_barrier()
        pltpu.sync_copy(cnt_v, acc_sh.at[row_v], add=True)
        plsc.subcore_barrier()
        crows = pl.ds(sid * 8, 8)

        @pl.when(jnp.logical_and(cid == 0, sid < 10))
        def _():
            pltpu.sync_copy(acc_sh.at[crows], cs_out.at[crows])

        @pl.when(jnp.logical_and(cid == 1, sid < 10))
        def _():
            pltpu.sync_copy(acc_sh.at[crows], cd_out.at[crows])

    @functools.partial(
        pl.kernel,
        out_type=jax.ShapeDtypeStruct((NCORE * NP, H), jnp.float32),
        mesh=mesh,
        scratch_types=[
            pltpu.VMEM_SHARED((NP, H), jnp.float32),
            pltpu.VMEM((K,), jnp.int32),
            pltpu.VMEM((K,), jnp.int32),
            pltpu.VMEM((K, H), jnp.float32),
        ],
    )
    def scatter_kernel(src_hbm, dst_hbm, el2, zf, s2_out,
                       acc_sh, src_v, dst_v, rows_v):
        cid = lax.axis_index("c")
        sid = lax.axis_index("s")
        r0 = sid * ROWS_PER_TILE
        rows = pl.ds(r0, ROWS_PER_TILE)
        pltpu.sync_copy(zf.at[rows], acc_sh.at[rows])
        plsc.subcore_barrier()
        e0 = sid * EPT
        off = cid * NP

        @pl.loop(0, CHUNKS)
        def _(j):
            b = e0 + j * K
            pltpu.sync_copy(src_hbm.at[pl.ds(b, K)], src_v)
            pltpu.sync_copy(dst_hbm.at[pl.ds(b, K)], dst_v)
            for k in range(K // 16):
                sl = pl.ds(k * 16, 16)
                src_v[sl] = src_v[sl] + off
            pltpu.sync_copy(el2.at[src_v], rows_v)
            pltpu.sync_copy(rows_v, acc_sh.at[dst_v], add=True)

        plsc.subcore_barrier()
        out_rows = pl.ds(off + r0, ROWS_PER_TILE)
        pltpu.sync_copy(acc_sh.at[rows], s2_out.at[out_rows])

    return degree_kernel, scatter_kernel


def _scale_body(feat_ref, cnt_ref, el2_ref):
    cnt = cnt_ref[...].astype(jnp.float32)
    norm = lax.rsqrt(jnp.maximum(cnt, 1.0))
    el2_ref[...] = feat_ref[...] * norm


def _final_body(feat_ref, slo_ref, shi_ref, cnt_ref, wm_ref, wn_ref,
                bmn_ref, bn_ref, out_ref):
    cnt = cnt_ref[...].astype(jnp.float32)
    norm = lax.rsqrt(jnp.maximum(cnt, 1.0))
    s = jnp.concatenate([slo_ref[...], shi_ref[...]], axis=1)
    t = norm * s
    f = feat_ref[...]
    out_ref[...] = (
        jnp.dot(t * f, wm_ref[...], preferred_element_type=jnp.float32)
        + jnp.dot(t + f, wn_ref[...], preferred_element_type=jnp.float32)
        + (norm * cnt) * bmn_ref[...]
        + bn_ref[...]
    )


def kernel(feat, edge_index, Wm, bm, Wn, bn):
    pad = jnp.full((EP - E,), N, jnp.int32)
    src = jnp.concatenate([edge_index[0].astype(jnp.int32), pad])
    dst = jnp.concatenate([edge_index[1].astype(jnp.int32), pad])
    feat_p = jnp.pad(feat, ((0, NP - N), (0, 0)))
    z80 = jnp.zeros((CROWS, H), jnp.int32)
    zf = jnp.zeros((NP, H), jnp.float32)

    degree_kernel, scatter_kernel = _sc_kernels()
    cs80, cd80 = degree_kernel(src, dst, z80)
    cs = cs80.reshape(NP, 1)
    cd = cd80.reshape(NP, 1)

    el2 = pl.pallas_call(
        _scale_body,
        grid=(GRID, NCORE),
        in_specs=[
            pl.BlockSpec((RB, H), lambda i, c: (i, c)),
            pl.BlockSpec((RB, 1), lambda i, c: (i, 0)),
        ],
        out_specs=pl.BlockSpec((RB, H), lambda i, c: (c * GRID + i, 0)),
        out_shape=jax.ShapeDtypeStruct((NCORE * NP, H), jnp.float32),
    )(feat_p, cs)

    s2 = scatter_kernel(src, dst, el2, zf)

    out = pl.pallas_call(
        _final_body,
        grid=(GRID,),
        in_specs=[
            pl.BlockSpec((RB, D), lambda i: (i, 0)),
            pl.BlockSpec((RB, H), lambda i: (i, 0)),
            pl.BlockSpec((RB, H), lambda i: (GRID + i, 0)),
            pl.BlockSpec((RB, 1), lambda i: (i, 0)),
            pl.BlockSpec((D, D), lambda i: (0, 0)),
            pl.BlockSpec((D, D), lambda i: (0, 0)),
            pl.BlockSpec((1, D), lambda i: (0, 0)),
            pl.BlockSpec((1, D), lambda i: (0, 0)),
        ],
        out_specs=pl.BlockSpec((RB, D), lambda i: (i, 0)),
        out_shape=jax.ShapeDtypeStruct((N, D), jnp.float32),
    )(feat_p, s2, s2, cd, Wm, Wn,
      (bm + bn).reshape(1, D), bn.reshape(1, D))

    return out


# speedup vs baseline: 4.5874x; 1.2217x over previous
import dataclasses
import functools

import jax
import jax.numpy as jnp
from jax import lax
from jax.experimental import pallas as pl
from jax.experimental.pallas import tpu as pltpu
from jax.experimental.pallas import tpu_sc as plsc

N = 10000
NP = 10240
E = 160000
EP = 163840
D = 256
H = 128
K = 512
KS = 320
NSUB = 16
NCORE = 2
CROWS = NP // H
ROWS_PER_TILE = NP // NSUB
EPT = EP // NSUB
CHUNKS = EPT // K
CHUNKS_S = EPT // KS
RB = 1280
GRID = NP // RB


@functools.lru_cache(maxsize=None)
def _sc_kernels():
    mesh = plsc.VectorSubcoreMesh(core_axis_name="c", subcore_axis_name="s")
    cp = pltpu.CompilerParams()
    if "needs_layout_passes" in pltpu.CompilerParams.__dataclass_fields__:
        cp = dataclasses.replace(cp, needs_layout_passes=False)

    @functools.partial(
        pl.kernel,
        out_type=(jax.ShapeDtypeStruct((CROWS, H), jnp.int32),
                  jax.ShapeDtypeStruct((CROWS, H), jnp.int32)),
        mesh=mesh,
        compiler_params=cp,
        scratch_types=[
            pltpu.VMEM_SHARED((CROWS, H), jnp.int32),
            pltpu.VMEM((CROWS, H), jnp.int32),
            pltpu.VMEM((K,), jnp.int32),
            pltpu.VMEM((CROWS,), jnp.int32),
        ],
    )
    def degree_kernel(src_hbm, dst_hbm, z80, cs_out, cd_out,
                      acc_sh, cnt_v, idx_v, row_v):
        cid = lax.axis_index("c")
        sid = lax.axis_index("s")
        pltpu.sync_copy(z80, cnt_v)
        for k in range(CROWS // 16):
            row_v[pl.ds(k * 16, 16)] = lax.iota(jnp.int32, 16) + k * 16

        @pl.when(sid == 0)
        def _():
            pltpu.sync_copy(z80, acc_sh)

        ones16 = jnp.ones((16,), jnp.int32)
        e0 = sid * EPT

        @pl.loop(0, CHUNKS)
        def _(j):
            b = pl.ds(e0 + j * K, K)

            @pl.when(cid == 0)
            def _():
                pltpu.sync_copy(src_hbm.at[b], idx_v)

            @pl.when(cid == 1)
            def _():
                pltpu.sync_copy(dst_hbm.at[b], idx_v)

            for k in range(K // 16):
                v = idx_v[pl.ds(k * 16, 16)]
                hi = lax.shift_right_logical(v, 7)
                lo = lax.bitwise_and(v, 127)
                plsc.addupdate_scatter(cnt_v, [hi, lo], ones16)

        plsc.subcore_barrier()
        pltpu.sync_copy(cnt_v, acc_sh.at[row_v], add=True)
        plsc.subcore_barrier()
        crows = pl.ds(sid * 8, 8)

        @pl.when(jnp.logical_and(cid == 0, sid < 10))
        def _():
            pltpu.sync_copy(acc_sh.at[crows], cs_out.at[crows])

        @pl.when(jnp.logical_and(cid == 1, sid < 10))
        def _():
            pltpu.sync_copy(acc_sh.at[crows], cd_out.at[crows])

    @functools.partial(
        pl.kernel,
        out_type=jax.ShapeDtypeStruct((NCORE * NP, H), jnp.float32),
        mesh=mesh,
        scratch_types=[
            pltpu.VMEM_SHARED((NP, H), jnp.float32),
            pltpu.VMEM((KS,), jnp.int32),
            pltpu.VMEM((KS,), jnp.int32),
            pltpu.VMEM((KS, H), jnp.float32),
            pltpu.SemaphoreType.DMA,
            pltpu.SemaphoreType.DMA,
            pltpu.SemaphoreType.DMA,
        ],
    )
    def scatter_kernel(src_hbm, dst_hbm, el2, zf, s2_out,
                       acc_sh, src_v, dst_v, rows_v, sem_a, sem_b, sem_s):
        cid = lax.axis_index("c")
        sid = lax.axis_index("s")
        r0 = sid * ROWS_PER_TILE
        rows = pl.ds(r0, ROWS_PER_TILE)
        pltpu.sync_copy(zf.at[rows], acc_sh.at[rows])
        plsc.subcore_barrier()
        e0 = sid * EPT
        off = cid * NP

        @pl.loop(0, CHUNKS_S)
        def _(j):
            @pl.when(j > 0)
            def _():
                pltpu.make_async_copy(rows_v, acc_sh.at[dst_v], sem_s).wait()

            b = e0 + j * KS
            ca = pltpu.async_copy(src_hbm.at[pl.ds(b, KS)], src_v, sem_a)
            cb = pltpu.async_copy(dst_hbm.at[pl.ds(b, KS)], dst_v, sem_b)
            ca.wait()
            cb.wait()
            for k in range(KS // 16):
                sl = pl.ds(k * 16, 16)
                src_v[sl] = src_v[sl] + off
            pltpu.sync_copy(el2.at[src_v], rows_v)
            pltpu.async_copy(rows_v, acc_sh.at[dst_v], sem_s, add=True)

        pltpu.make_async_copy(rows_v, acc_sh.at[dst_v], sem_s).wait()
        plsc.subcore_barrier()
        out_rows = pl.ds(off + r0, ROWS_PER_TILE)
        pltpu.sync_copy(acc_sh.at[rows], s2_out.at[out_rows])

    return degree_kernel, scatter_kernel


def _scale_body(feat_ref, cnt_ref, el2_ref):
    cnt = cnt_ref[...].astype(jnp.float32)
    norm = lax.rsqrt(jnp.maximum(cnt, 1.0))
    el2_ref[...] = feat_ref[...] * norm


def _final_body(feat_ref, slo_ref, shi_ref, cnt_ref, wm_ref, wn_ref,
                bmn_ref, bn_ref, out_ref):
    cnt = cnt_ref[...].astype(jnp.float32)
    norm = lax.rsqrt(jnp.maximum(cnt, 1.0))
    s = jnp.concatenate([slo_ref[...], shi_ref[...]], axis=1)
    t = norm * s
    f = feat_ref[...]
    out_ref[...] = (
        jnp.dot(t * f, wm_ref[...], preferred_element_type=jnp.float32)
        + jnp.dot(t + f, wn_ref[...], preferred_element_type=jnp.float32)
        + (norm * cnt) * bmn_ref[...]
        + bn_ref[...]
    )


def kernel(feat, edge_index, Wm, bm, Wn, bn):
    pad = jnp.full((EP - E,), N, jnp.int32)
    src = jnp.concatenate([edge_index[0].astype(jnp.int32), pad])
    dst = jnp.concatenate([edge_index[1].astype(jnp.int32), pad])
    feat_p = jnp.pad(feat, ((0, NP - N), (0, 0)))
    z80 = jnp.zeros((CROWS, H), jnp.int32)
    zf = jnp.zeros((NP, H), jnp.float32)

    degree_kernel, scatter_kernel = _sc_kernels()
    cs80, cd80 = degree_kernel(src, dst, z80)
    cs = cs80.reshape(NP, 1)
    cd = cd80.reshape(NP, 1)

    el2 = pl.pallas_call(
        _scale_body,
        grid=(GRID, NCORE),
        in_specs=[
            pl.BlockSpec((RB, H), lambda i, c: (i, c)),
            pl.BlockSpec((RB, 1), lambda i, c: (i, 0)),
        ],
        out_specs=pl.BlockSpec((RB, H), lambda i, c: (c * GRID + i, 0)),
        out_shape=jax.ShapeDtypeStruct((NCORE * NP, H), jnp.float32),
    )(feat_p, cs)

    s2 = scatter_kernel(src, dst, el2, zf)

    out = pl.pallas_call(
        _final_body,
        grid=(GRID,),
        in_specs=[
            pl.BlockSpec((RB, D), lambda i: (i, 0)),
            pl.BlockSpec((RB, H), lambda i: (i, 0)),
            pl.BlockSpec((RB, H), lambda i: (GRID + i, 0)),
            pl.BlockSpec((RB, 1), lambda i: (i, 0)),
            pl.BlockSpec((D, D), lambda i: (0, 0)),
            pl.BlockSpec((D, D), lambda i: (0, 0)),
            pl.BlockSpec((1, D), lambda i: (0, 0)),
            pl.BlockSpec((1, D), lambda i: (0, 0)),
        ],
        out_specs=pl.BlockSpec((RB, D), lambda i: (i, 0)),
        out_shape=jax.ShapeDtypeStruct((N, D), jnp.float32),
    )(feat_p, s2, s2, cd, Wm, Wn,
      (bm + bn).reshape(1, D), bn.reshape(1, D))

    return out


# speedup vs baseline: 4.9932x; 1.0885x over previous
import dataclasses
import functools

import jax
import jax.numpy as jnp
from jax import lax
from jax.experimental import pallas as pl
from jax.experimental.pallas import tpu as pltpu
from jax.experimental.pallas import tpu_sc as plsc

N = 10000
NP = 10240
E = 160000
EP = 163840
D = 256
H = 128
K = 512
KS = 160
NSUB = 16
NCORE = 2
CROWS = NP // H
ROWS_PER_TILE = NP // NSUB
EPT = EP // NSUB
CHUNKS = EPT // K
CHUNKS_S = EPT // KS
RB = 1280
GRID = NP // RB


@functools.lru_cache(maxsize=None)
def _sc_kernels():
    mesh = plsc.VectorSubcoreMesh(core_axis_name="c", subcore_axis_name="s")
    cp = pltpu.CompilerParams()
    if "needs_layout_passes" in pltpu.CompilerParams.__dataclass_fields__:
        cp = dataclasses.replace(cp, needs_layout_passes=False)

    @functools.partial(
        pl.kernel,
        out_type=(jax.ShapeDtypeStruct((CROWS, H), jnp.int32),
                  jax.ShapeDtypeStruct((CROWS, H), jnp.int32)),
        mesh=mesh,
        compiler_params=cp,
        scratch_types=[
            pltpu.VMEM_SHARED((CROWS, H), jnp.int32),
            pltpu.VMEM((CROWS, H), jnp.int32),
            pltpu.VMEM((K,), jnp.int32),
            pltpu.VMEM((CROWS,), jnp.int32),
        ],
    )
    def degree_kernel(src_hbm, dst_hbm, z80, cs_out, cd_out,
                      acc_sh, cnt_v, idx_v, row_v):
        cid = lax.axis_index("c")
        sid = lax.axis_index("s")
        pltpu.sync_copy(z80, cnt_v)
        for k in range(CROWS // 16):
            row_v[pl.ds(k * 16, 16)] = lax.iota(jnp.int32, 16) + k * 16

        @pl.when(sid == 0)
        def _():
            pltpu.sync_copy(z80, acc_sh)

        ones16 = jnp.ones((16,), jnp.int32)
        e0 = sid * EPT

        @pl.loop(0, CHUNKS)
        def _(j):
            b = pl.ds(e0 + j * K, K)

            @pl.when(cid == 0)
            def _():
                pltpu.sync_copy(src_hbm.at[b], idx_v)

            @pl.when(cid == 1)
            def _():
                pltpu.sync_copy(dst_hbm.at[b], idx_v)

            for k in range(K // 16):
                v = idx_v[pl.ds(k * 16, 16)]
                hi = lax.shift_right_logical(v, 7)
                lo = lax.bitwise_and(v, 127)
                plsc.addupdate_scatter(cnt_v, [hi, lo], ones16)

        plsc.subcore_barrier()
        pltpu.sync_copy(cnt_v, acc_sh.at[row_v], add=True)
        plsc.subcore_barrier()
        crows = pl.ds(sid * 8, 8)

        @pl.when(jnp.logical_and(cid == 0, sid < 10))
        def _():
            pltpu.sync_copy(acc_sh.at[crows], cs_out.at[crows])

        @pl.when(jnp.logical_and(cid == 1, sid < 10))
        def _():
            pltpu.sync_copy(acc_sh.at[crows], cd_out.at[crows])

    @functools.partial(
        pl.kernel,
        out_type=jax.ShapeDtypeStruct((NCORE * NP, H), jnp.float32),
        mesh=mesh,
        scratch_types=[
            pltpu.VMEM_SHARED((NP, H), jnp.float32),
            pltpu.VMEM((KS,), jnp.int32),
            pltpu.VMEM((KS,), jnp.int32),
            pltpu.VMEM((KS,), jnp.int32),
            pltpu.VMEM((KS,), jnp.int32),
            pltpu.VMEM((KS,), jnp.int32),
            pltpu.VMEM((KS,), jnp.int32),
            pltpu.VMEM((KS, H), jnp.float32),
            pltpu.VMEM((KS, H), jnp.float32),
            pltpu.SemaphoreType.DMA,
            pltpu.SemaphoreType.DMA,
            pltpu.SemaphoreType.DMA,
            pltpu.SemaphoreType.DMA,
            pltpu.SemaphoreType.DMA,
            pltpu.SemaphoreType.DMA,
            pltpu.SemaphoreType.DMA,
            pltpu.SemaphoreType.DMA,
        ],
    )
    def scatter_kernel(src_hbm, dst_hbm, el2, zf, s2_out, acc_sh,
                       src0, src1, ld0, ld1, sd0, sd1, rows0, rows1,
                       sa0, sa1, sb0, sb1, sg0, sg1, ss0, ss1):
        cid = lax.axis_index("c")
        sid = lax.axis_index("s")
        r0 = sid * ROWS_PER_TILE
        rows = pl.ds(r0, ROWS_PER_TILE)
        pltpu.sync_copy(zf.at[rows], acc_sh.at[rows])
        plsc.subcore_barrier()
        e0 = sid * EPT
        off = cid * NP

        src_b = (src0, src1)
        ld_b = (ld0, ld1)
        sd_b = (sd0, sd1)
        rows_b = (rows0, rows1)
        sa_b = (sa0, sa1)
        sb_b = (sb0, sb1)
        sg_b = (sg0, sg1)
        ss_b = (ss0, ss1)

        def issue_idx(j, b):
            e = e0 + j * KS
            pltpu.async_copy(src_hbm.at[pl.ds(e, KS)], src_b[b], sa_b[b])
            pltpu.async_copy(dst_hbm.at[pl.ds(e, KS)], ld_b[b], sb_b[b])

        def wait_idx(j, b):
            e = e0 + j * KS
            pltpu.make_async_copy(
                src_hbm.at[pl.ds(e, KS)], src_b[b], sa_b[b]).wait()
            pltpu.make_async_copy(
                dst_hbm.at[pl.ds(e, KS)], ld_b[b], sb_b[b]).wait()

        def prep(b):
            for k in range(KS // 16):
                sl = pl.ds(k * 16, 16)
                src_b[b][sl] = src_b[b][sl] + off
                sd_b[b][sl] = ld_b[b][sl]

        def issue_gather(b):
            pltpu.async_copy(el2.at[src_b[b]], rows_b[b], sg_b[b])

        def wait_gather(b):
            pltpu.make_async_copy(el2.at[src_b[b]], rows_b[b], sg_b[b]).wait()

        def issue_scatter(b):
            pltpu.async_copy(rows_b[b], acc_sh.at[sd_b[b]], ss_b[b], add=True)

        def wait_scatter(b):
            pltpu.make_async_copy(
                rows_b[b], acc_sh.at[sd_b[b]], ss_b[b]).wait()

        issue_idx(0, 0)
        issue_idx(1, 1)
        wait_idx(0, 0)
        prep(0)
        issue_gather(0)
        wait_idx(1, 1)
        prep(1)
        wait_gather(0)
        issue_scatter(0)
        issue_gather(1)
        issue_idx(2, 0)

        @pl.loop(0, (CHUNKS_S - 2) // 2)
        def _(g):
            j0 = 2 * g + 2

            wait_idx(j0, 0)
            wait_scatter(0)
            prep(0)
            wait_gather(1)
            issue_scatter(1)
            issue_gather(0)
            issue_idx(j0 + 1, 1)

            wait_idx(j0 + 1, 1)
            wait_scatter(1)
            prep(1)
            wait_gather(0)
            issue_scatter(0)
            issue_gather(1)

            @pl.when(j0 + 2 < CHUNKS_S)
            def _():
                issue_idx(j0 + 2, 0)

        wait_gather(1)
        issue_scatter(1)
        wait_scatter(0)
        wait_scatter(1)
        plsc.subcore_barrier()
        out_rows = pl.ds(off + r0, ROWS_PER_TILE)
        pltpu.sync_copy(acc_sh.at[rows], s2_out.at[out_rows])

    return degree_kernel, scatter_kernel


def _scale_body(feat_ref, cnt_ref, el2_ref):
    cnt = cnt_ref[...].astype(jnp.float32)
    norm = lax.rsqrt(jnp.maximum(cnt, 1.0))
    el2_ref[...] = feat_ref[...] * norm


def _final_body(feat_ref, slo_ref, shi_ref, cnt_ref, wm_ref, wn_ref,
                bmn_ref, bn_ref, out_ref):
    cnt = cnt_ref[...].astype(jnp.float32)
    norm = lax.rsqrt(jnp.maximum(cnt, 1.0))
    s = jnp.concatenate([slo_ref[...], shi_ref[...]], axis=1)
    t = norm * s
    f = feat_ref[...]
    out_ref[...] = (
        jnp.dot(t * f, wm_ref[...], preferred_element_type=jnp.float32)
        + jnp.dot(t + f, wn_ref[...], preferred_element_type=jnp.float32)
        + (norm * cnt) * bmn_ref[...]
        + bn_ref[...]
    )


def kernel(feat, edge_index, Wm, bm, Wn, bn):
    pad = jnp.full((EP - E,), N, jnp.int32)
    src = jnp.concatenate([edge_index[0].astype(jnp.int32), pad])
    dst = jnp.concatenate([edge_index[1].astype(jnp.int32), pad])
    feat_p = jnp.pad(feat, ((0, NP - N), (0, 0)))
    z80 = jnp.zeros((CROWS, H), jnp.int32)
    zf = jnp.zeros((NP, H), jnp.float32)

    degree_kernel, scatter_kernel = _sc_kernels()
    cs80, cd80 = degree_kernel(src, dst, z80)
    cs = cs80.reshape(NP, 1)
    cd = cd80.reshape(NP, 1)

    el2 = pl.pallas_call(
        _scale_body,
        grid=(GRID, NCORE),
        in_specs=[
            pl.BlockSpec((RB, H), lambda i, c: (i, c)),
            pl.BlockSpec((RB, 1), lambda i, c: (i, 0)),
        ],
        out_specs=pl.BlockSpec((RB, H), lambda i, c: (c * GRID + i, 0)),
        out_shape=jax.ShapeDtypeStruct((NCORE * NP, H), jnp.float32),
    )(feat_p, cs)

    s2 = scatter_kernel(src, dst, el2, zf)

    out = pl.pallas_call(
        _final_body,
        grid=(GRID,),
        in_specs=[
            pl.BlockSpec((RB, D), lambda i: (i, 0)),
            pl.BlockSpec((RB, H), lambda i: (i, 0)),
            pl.BlockSpec((RB, H), lambda i: (GRID + i, 0)),
            pl.BlockSpec((RB, 1), lambda i: (i, 0)),
            pl.BlockSpec((D, D), lambda i: (0, 0)),
            pl.BlockSpec((D, D), lambda i: (0, 0)),
            pl.BlockSpec((1, D), lambda i: (0, 0)),
            pl.BlockSpec((1, D), lambda i: (0, 0)),
        ],
        out_specs=pl.BlockSpec((RB, D), lambda i: (i, 0)),
        out_shape=jax.ShapeDtypeStruct((N, D), jnp.float32),
    )(feat_p, s2, s2, cd, Wm, Wn,
      (bm + bn).reshape(1, D), bn.reshape(1, D))

    return out
